# baseline (speedup 1.0000x reference)
import functools

import jax
import jax.numpy as jnp
from jax import lax
from jax.experimental import pallas as pl
from jax.experimental.pallas import tpu as pltpu
from jax.experimental.pallas import tpu_sc as plsc

NUM_ROWS = 1000000
EMBED_DIM = 32
BATCH = 16384

_INFO = plsc.get_sparse_core_info()
_NC = _INFO.num_cores
_NS = _INFO.num_subcores
_NW = _NC * _NS
_BPW = BATCH // _NW
_CHUNK = 128
_L = 16


def _rsqrt16(x):
    i = lax.bitcast_convert_type(x, jnp.int32)
    i = jnp.int32(0x5F3759DF) - (i >> 1)
    y = lax.bitcast_convert_type(i, jnp.float32)
    for _ in range(3):
        y = y * (jnp.float32(1.5) - jnp.float32(0.5) * x * y * y)
    return y


def _build_indices(idb, ixb):
    def block(b, carry):
        id16 = idb[pl.ds(b * _L, _L)]
        for d in range(EMBED_DIM):
            ixb[d, pl.ds(b * _L, _L)] = id16 + d * NUM_ROWS
        return carry
    lax.fori_loop(0, _BPW // _L, block, 0)


def _fire_gathers(tab, ixb, gbuf, sem):
    cps = []
    for d in range(EMBED_DIM):
        for rb in range(_BPW // _CHUNK):
            cps.append(pltpu.async_copy(
                tab.at[ixb.at[d, pl.ds(rb * _CHUNK, _CHUNK)]],
                gbuf.at[d, pl.ds(rb * _CHUNK, _CHUNK)], sem))
    return cps


def _normalize(gbuf):
    def block(b, carry):
        sl = pl.ds(b * _L, _L)
        acc = jnp.zeros((_L,), jnp.float32)
        for d in range(EMBED_DIM):
            v = gbuf[d, sl]
            acc = acc + v * v
        scale = _rsqrt16(jnp.maximum(acc, jnp.float32(1e-12)))
        for d in range(EMBED_DIM):
            gbuf[d, sl] = gbuf[d, sl] * scale
        return carry
    lax.fori_loop(0, _BPW // _L, block, 0)


def _body(uid_hbm, iid_hbm, utab_hbm, itab_hbm, uout_hbm, iout_hbm,
          uidb, iidb, uix, iix, ugb, igb, usem, isem):
    wid = lax.axis_index("s") * _NC + lax.axis_index("c")
    base = wid * _BPW

    pltpu.sync_copy(uid_hbm.at[pl.ds(base, _BPW)], uidb)
    pltpu.sync_copy(iid_hbm.at[pl.ds(base, _BPW)], iidb)

    _build_indices(uidb, uix)
    ucp = _fire_gathers(utab_hbm, uix, ugb, usem)
    _build_indices(iidb, iix)
    icp = _fire_gathers(itab_hbm, iix, igb, isem)

    for c in ucp:
        c.wait()
    _normalize(ugb)
    pltpu.sync_copy(ugb, uout_hbm.at[:, pl.ds(base, _BPW)])

    for c in icp:
        c.wait()
    _normalize(igb)
    pltpu.sync_copy(igb, iout_hbm.at[:, pl.ds(base, _BPW)])


@functools.partial(
    pl.kernel,
    out_type=(
        jax.ShapeDtypeStruct((EMBED_DIM, BATCH), jnp.float32),
        jax.ShapeDtypeStruct((EMBED_DIM, BATCH), jnp.float32),
    ),
    mesh=plsc.VectorSubcoreMesh(core_axis_name="c", subcore_axis_name="s"),
    compiler_params=pltpu.CompilerParams(needs_layout_passes=False),
    scratch_types=[
        pltpu.VMEM((_BPW,), jnp.int32),
        pltpu.VMEM((_BPW,), jnp.int32),
        pltpu.VMEM((EMBED_DIM, _BPW), jnp.int32),
        pltpu.VMEM((EMBED_DIM, _BPW), jnp.int32),
        pltpu.VMEM((EMBED_DIM, _BPW), jnp.float32),
        pltpu.VMEM((EMBED_DIM, _BPW), jnp.float32),
        pltpu.SemaphoreType.DMA,
        pltpu.SemaphoreType.DMA,
    ],
)
def _sc_lookup_normalize(uid_hbm, iid_hbm, utab_hbm, itab_hbm,
                         uout_hbm, iout_hbm,
                         uidb, iidb, uix, iix, ugb, igb, usem, isem):
    _body(uid_hbm, iid_hbm, utab_hbm, itab_hbm, uout_hbm, iout_hbm,
          uidb, iidb, uix, iix, ugb, igb, usem, isem)


def kernel(user_id, item_id, user_table, item_table):
    utab1 = user_table.T.reshape(NUM_ROWS * EMBED_DIM)
    itab1 = item_table.T.reshape(NUM_ROWS * EMBED_DIM)
    uoT, ioT = _sc_lookup_normalize(user_id, item_id, utab1, itab1)
    return (uoT.T, ioT.T)


# speedup vs baseline: 1.0018x; 1.0018x over previous
import functools

import jax
import jax.numpy as jnp
from jax import lax
from jax.experimental import pallas as pl
from jax.experimental.pallas import tpu as pltpu
from jax.experimental.pallas import tpu_sc as plsc

NUM_ROWS = 1000000
EMBED_DIM = 32
BATCH = 16384

_INFO = plsc.get_sparse_core_info()
_NC = _INFO.num_cores
_NS = _INFO.num_subcores
_NW = _NC * _NS
_BPW = BATCH // _NW
_CHUNK = 128
_L = 16


def _rsqrt16(x):
    i = lax.bitcast_convert_type(x, jnp.int32)
    i = jnp.int32(0x5F3759DF) - (i >> 1)
    y = lax.bitcast_convert_type(i, jnp.float32)
    for _ in range(3):
        y = y * (jnp.float32(1.5) - jnp.float32(0.5) * x * y * y)
    return y


def _fire_gathers(tab, idb, gbuf, sem):
    _UNROLL = 4

    def dloop(d, carry):
        dbase = d * NUM_ROWS

        def block(b8, carry2):
            for u in range(_UNROLL):
                b = b8 * _UNROLL + u
                idx16 = idb[pl.ds(b * _L, _L)] + dbase
                pltpu.async_copy(
                    tab.at[idx16], gbuf.at[d, pl.ds(b * _L, _L)], sem)
            return carry2

        lax.fori_loop(0, _BPW // _L // _UNROLL, block, 0)
        return carry

    lax.fori_loop(0, EMBED_DIM, dloop, 0)


def _drain(gbuf, dummy_src, sem):
    pltpu.make_async_copy(dummy_src, gbuf, sem).wait()


def _normalize(gbuf):
    def block(b, carry):
        sl = pl.ds(b * _L, _L)
        acc = jnp.zeros((_L,), jnp.float32)
        for d in range(EMBED_DIM):
            v = gbuf[d, sl]
            acc = acc + v * v
        scale = _rsqrt16(jnp.maximum(acc, jnp.float32(1e-12)))
        for d in range(EMBED_DIM):
            gbuf[d, sl] = gbuf[d, sl] * scale
        return carry
    lax.fori_loop(0, _BPW // _L, block, 0)


def _body(uid_hbm, iid_hbm, utab_hbm, itab_hbm, uout_hbm, iout_hbm,
          uidb, iidb, ugb, igb, usem, isem):
    wid = lax.axis_index("s") * _NC + lax.axis_index("c")
    base = wid * _BPW

    pltpu.sync_copy(uid_hbm.at[pl.ds(base, _BPW)], uidb)
    pltpu.sync_copy(iid_hbm.at[pl.ds(base, _BPW)], iidb)

    _fire_gathers(utab_hbm, uidb, ugb, usem)
    _fire_gathers(itab_hbm, iidb, igb, isem)

    _drain(ugb, uout_hbm.at[:, pl.ds(0, _BPW)], usem)
    _normalize(ugb)
    pltpu.sync_copy(ugb, uout_hbm.at[:, pl.ds(base, _BPW)])

    _drain(igb, iout_hbm.at[:, pl.ds(0, _BPW)], isem)
    _normalize(igb)
    pltpu.sync_copy(igb, iout_hbm.at[:, pl.ds(base, _BPW)])


@functools.partial(
    pl.kernel,
    out_type=(
        jax.ShapeDtypeStruct((EMBED_DIM, BATCH), jnp.float32),
        jax.ShapeDtypeStruct((EMBED_DIM, BATCH), jnp.float32),
    ),
    mesh=plsc.VectorSubcoreMesh(core_axis_name="c", subcore_axis_name="s"),
    compiler_params=pltpu.CompilerParams(needs_layout_passes=False),
    scratch_types=[
        pltpu.VMEM((_BPW,), jnp.int32),
        pltpu.VMEM((_BPW,), jnp.int32),
        pltpu.VMEM((EMBED_DIM, _BPW), jnp.float32),
        pltpu.VMEM((EMBED_DIM, _BPW), jnp.float32),
        pltpu.SemaphoreType.DMA,
        pltpu.SemaphoreType.DMA,
    ],
)
def _sc_lookup_normalize(uid_hbm, iid_hbm, utab_hbm, itab_hbm,
                         uout_hbm, iout_hbm,
                         uidb, iidb, ugb, igb, usem, isem):
    _body(uid_hbm, iid_hbm, utab_hbm, itab_hbm, uout_hbm, iout_hbm,
          uidb, iidb, ugb, igb, usem, isem)


def kernel(user_id, item_id, user_table, item_table):
    utab1 = user_table.T.reshape(NUM_ROWS * EMBED_DIM)
    itab1 = item_table.T.reshape(NUM_ROWS * EMBED_DIM)
    uoT, ioT = _sc_lookup_normalize(user_id, item_id, utab1, itab1)
    return (uoT.T, ioT.T)


# speedup vs baseline: 5.4643x; 5.4543x over previous
import functools

import jax
import jax.numpy as jnp
from jax import lax
from jax.experimental import pallas as pl
from jax.experimental.pallas import tpu as pltpu
from jax.experimental.pallas import tpu_sc as plsc

NUM_ROWS = 1000000
EMBED_DIM = 32
BATCH = 16384
GPR = 128 // EMBED_DIM
NUM_GROUPS = NUM_ROWS // GPR

_INFO = plsc.get_sparse_core_info()
_NC = _INFO.num_cores
_NS = _INFO.num_subcores
_NW = _NC * _NS
_BPW = BATCH // _NW
_CHUNK = 128
_NCHUNK = _BPW // _CHUNK
_L = 16


def _rsqrt16(x):
    i = lax.bitcast_convert_type(x, jnp.int32)
    i = jnp.int32(0x5F3759DF) - (i >> 1)
    y = lax.bitcast_convert_type(i, jnp.float32)
    for _ in range(3):
        y = y * (jnp.float32(1.5) - jnp.float32(0.5) * x * y * y)
    return y


def _process_chunk(j, idb, buf, outb):
    lane = lax.iota(jnp.int32, _L)

    def group(g, carry):
        pos = j * _CHUNK + g * _L
        id16 = idb[pl.ds(pos, _L)]
        cbase = (id16 & (GPR - 1)) << 5
        ridx = g * _L + lane
        acc = jnp.zeros((_L,), jnp.float32)
        for d in range(EMBED_DIM):
            v = plsc.load_gather(buf, [ridx, cbase + d])
            acc = acc + v * v
        scale = _rsqrt16(jnp.maximum(acc, jnp.float32(1e-12)))
        for d in range(EMBED_DIM):
            v = plsc.load_gather(buf, [ridx, cbase + d])
            outb[d, pl.ds(pos, _L)] = v * scale
        return carry

    lax.fori_loop(0, _CHUNK // _L, group, 0)


def _group_indices(idb, gix):
    def step(k, carry):
        gix[pl.ds(k * _L, _L)] = idb[pl.ds(k * _L, _L)] >> 2
        return carry
    lax.fori_loop(0, _BPW // _L, step, 0)


def _body(uid_hbm, iid_hbm, utab_hbm, itab_hbm, uout_hbm, iout_hbm,
          uidb, iidb, ugix, igix, ub0, ub1, ib0, ib1, uoutb, ioutb,
          us0, us1, is0, is1):
    wid = lax.axis_index("s") * _NC + lax.axis_index("c")
    base = wid * _BPW

    pltpu.sync_copy(uid_hbm.at[pl.ds(base, _BPW)], uidb)
    pltpu.sync_copy(iid_hbm.at[pl.ds(base, _BPW)], iidb)
    _group_indices(uidb, ugix)
    _group_indices(iidb, igix)

    ubufs, usems = (ub0, ub1), (us0, us1)
    ibufs, isems = (ib0, ib1), (is0, is1)

    def fire(tab, gix, bufs, sems, j):
        return pltpu.async_copy(
            tab.at[gix.at[pl.ds(j * _CHUNK, _CHUNK)]], bufs[j % 2], sems[j % 2])

    ucp = [fire(utab_hbm, ugix, ubufs, usems, 0),
           fire(utab_hbm, ugix, ubufs, usems, 1)]
    icp = [fire(itab_hbm, igix, ibufs, isems, 0),
           fire(itab_hbm, igix, ibufs, isems, 1)]

    for j in range(_NCHUNK):
        ucp[j].wait()
        _process_chunk(j, uidb, ubufs[j % 2], uoutb)
        if j + 2 < _NCHUNK:
            ucp.append(fire(utab_hbm, ugix, ubufs, usems, j + 2))
    pltpu.sync_copy(uoutb, uout_hbm.at[:, pl.ds(base, _BPW)])

    for j in range(_NCHUNK):
        icp[j].wait()
        _process_chunk(j, iidb, ibufs[j % 2], ioutb)
        if j + 2 < _NCHUNK:
            icp.append(fire(itab_hbm, igix, ibufs, isems, j + 2))
    pltpu.sync_copy(ioutb, iout_hbm.at[:, pl.ds(base, _BPW)])


@functools.partial(
    pl.kernel,
    out_type=(
        jax.ShapeDtypeStruct((EMBED_DIM, BATCH), jnp.float32),
        jax.ShapeDtypeStruct((EMBED_DIM, BATCH), jnp.float32),
    ),
    mesh=plsc.VectorSubcoreMesh(core_axis_name="c", subcore_axis_name="s"),
    compiler_params=pltpu.CompilerParams(needs_layout_passes=False),
    scratch_types=[
        pltpu.VMEM((_BPW,), jnp.int32),
        pltpu.VMEM((_BPW,), jnp.int32),
        pltpu.VMEM((_BPW,), jnp.int32),
        pltpu.VMEM((_BPW,), jnp.int32),
        pltpu.VMEM((_CHUNK, 128), jnp.float32),
        pltpu.VMEM((_CHUNK, 128), jnp.float32),
        pltpu.VMEM((_CHUNK, 128), jnp.float32),
        pltpu.VMEM((_CHUNK, 128), jnp.float32),
        pltpu.VMEM((EMBED_DIM, _BPW), jnp.float32),
        pltpu.VMEM((EMBED_DIM, _BPW), jnp.float32),
        pltpu.SemaphoreType.DMA,
        pltpu.SemaphoreType.DMA,
        pltpu.SemaphoreType.DMA,
        pltpu.SemaphoreType.DMA,
    ],
)
def _sc_lookup_normalize(uid_hbm, iid_hbm, utab_hbm, itab_hbm,
                         uout_hbm, iout_hbm,
                         uidb, iidb, ugix, igix, ub0, ub1, ib0, ib1,
                         uoutb, ioutb, us0, us1, is0, is1):
    _body(uid_hbm, iid_hbm, utab_hbm, itab_hbm, uout_hbm, iout_hbm,
          uidb, iidb, ugix, igix, ub0, ub1, ib0, ib1, uoutb, ioutb,
          us0, us1, is0, is1)


def kernel(user_id, item_id, user_table, item_table):
    utab2 = user_table.reshape(NUM_GROUPS, 128)
    itab2 = item_table.reshape(NUM_GROUPS, 128)
    uoT, ioT = _sc_lookup_normalize(user_id, item_id, utab2, itab2)
    return (uoT.T, ioT.T)


# speedup vs baseline: 5.9504x; 1.0889x over previous
import functools

import jax
import jax.numpy as jnp
from jax import lax
from jax.experimental import pallas as pl
from jax.experimental.pallas import tpu as pltpu
from jax.experimental.pallas import tpu_sc as plsc

NUM_ROWS = 1000000
EMBED_DIM = 32
BATCH = 16384
GPR = 128 // EMBED_DIM
NUM_GROUPS = NUM_ROWS // GPR

_INFO = plsc.get_sparse_core_info()
_NC = _INFO.num_cores
_NS = _INFO.num_subcores
_NW = _NC * _NS
_BPW = BATCH // _NW
_CHUNK = 128
_NCHUNK = _BPW // _CHUNK
_L = 16


def _rsqrt16(x):
    i = lax.bitcast_convert_type(x, jnp.int32)
    i = jnp.int32(0x5F3759DF) - (i >> 1)
    y = lax.bitcast_convert_type(i, jnp.float32)
    for _ in range(3):
        y = y * (jnp.float32(1.5) - jnp.float32(0.5) * x * y * y)
    return y


def _process_chunk(j, idb, buf, outb):
    lane = lax.iota(jnp.int32, _L)

    def group(g, carry):
        pos = j * _CHUNK + g * _L
        id16 = idb[pl.ds(pos, _L)]
        cbase = (id16 & (GPR - 1)) << 5
        ridx = g * _L + lane
        acc = jnp.zeros((_L,), jnp.float32)
        for d in range(EMBED_DIM):
            v = plsc.load_gather(buf, [ridx, cbase + d])
            acc = acc + v * v
        scale = _rsqrt16(jnp.maximum(acc, jnp.float32(1e-12)))
        for d in range(EMBED_DIM):
            v = plsc.load_gather(buf, [ridx, cbase + d])
            outb[d, pl.ds(pos, _L)] = v * scale
        return carry

    lax.fori_loop(0, _CHUNK // _L, group, 0)


def _group_indices(idb, gix):
    def step(k, carry):
        gix[pl.ds(k * _L, _L)] = idb[pl.ds(k * _L, _L)] >> 2
        return carry
    lax.fori_loop(0, _BPW // _L, step, 0)


def _body(uid_hbm, iid_hbm, utab_hbm, itab_hbm, uout_hbm, iout_hbm,
          uidb, iidb, ugix, igix, ub0, ub1, ib0, ib1, uoutb, ioutb,
          us0, us1, is0, is1):
    wid = lax.axis_index("s") * _NC + lax.axis_index("c")
    base = wid * _BPW

    pltpu.sync_copy(uid_hbm.at[pl.ds(base, _BPW)], uidb)
    pltpu.sync_copy(iid_hbm.at[pl.ds(base, _BPW)], iidb)
    _group_indices(uidb, ugix)
    _group_indices(iidb, igix)

    ubufs, usems = (ub0, ub1), (us0, us1)
    ibufs, isems = (ib0, ib1), (is0, is1)

    def fire(tab, gix, bufs, sems, j):
        return pltpu.async_copy(
            tab.at[gix.at[pl.ds(j * _CHUNK, _CHUNK)]], bufs[j % 2], sems[j % 2])

    ucp = [fire(utab_hbm, ugix, ubufs, usems, 0),
           fire(utab_hbm, ugix, ubufs, usems, 1)]
    icp = [fire(itab_hbm, igix, ibufs, isems, 0),
           fire(itab_hbm, igix, ibufs, isems, 1)]

    for j in range(_NCHUNK):
        ucp[j].wait()
        _process_chunk(j, uidb, ubufs[j % 2], uoutb)
        if j + 2 < _NCHUNK:
            ucp.append(fire(utab_hbm, ugix, ubufs, usems, j + 2))
    pltpu.sync_copy(uoutb, uout_hbm.at[:, pl.ds(base, _BPW)])

    for j in range(_NCHUNK):
        icp[j].wait()
        _process_chunk(j, iidb, ibufs[j % 2], ioutb)
        if j + 2 < _NCHUNK:
            icp.append(fire(itab_hbm, igix, ibufs, isems, j + 2))
    pltpu.sync_copy(ioutb, iout_hbm.at[:, pl.ds(base, _BPW)])


@functools.partial(
    pl.kernel,
    out_type=(
        jax.ShapeDtypeStruct((EMBED_DIM, BATCH), jnp.float32),
        jax.ShapeDtypeStruct((EMBED_DIM, BATCH), jnp.float32),
    ),
    mesh=plsc.VectorSubcoreMesh(core_axis_name="c", subcore_axis_name="s"),
    compiler_params=pltpu.CompilerParams(needs_layout_passes=False),
    scratch_types=[
        pltpu.VMEM((_BPW,), jnp.int32),
        pltpu.VMEM((_BPW,), jnp.int32),
        pltpu.VMEM((_BPW,), jnp.int32),
        pltpu.VMEM((_BPW,), jnp.int32),
        pltpu.VMEM((_CHUNK, 128), jnp.float32),
        pltpu.VMEM((_CHUNK, 128), jnp.float32),
        pltpu.VMEM((_CHUNK, 128), jnp.float32),
        pltpu.VMEM((_CHUNK, 128), jnp.float32),
        pltpu.VMEM((EMBED_DIM, _BPW), jnp.float32),
        pltpu.VMEM((EMBED_DIM, _BPW), jnp.float32),
        pltpu.SemaphoreType.DMA,
        pltpu.SemaphoreType.DMA,
        pltpu.SemaphoreType.DMA,
        pltpu.SemaphoreType.DMA,
    ],
)
def _sc_lookup_normalize(uid_hbm, iid_hbm, utab_hbm, itab_hbm,
                         uout_hbm, iout_hbm,
                         uidb, iidb, ugix, igix, ub0, ub1, ib0, ib1,
                         uoutb, ioutb, us0, us1, is0, is1):
    _body(uid_hbm, iid_hbm, utab_hbm, itab_hbm, uout_hbm, iout_hbm,
          uidb, iidb, ugix, igix, ub0, ub1, ib0, ib1, uoutb, ioutb,
          us0, us1, is0, is1)


def _group_view(table):
    t3 = table.T.reshape(EMBED_DIM, NUM_GROUPS, GPR)
    t3 = lax.optimization_barrier(t3)
    return t3.transpose(1, 2, 0).reshape(NUM_GROUPS, 128)


def kernel(user_id, item_id, user_table, item_table):
    utab2 = _group_view(user_table)
    itab2 = _group_view(item_table)
    uoT, ioT = _sc_lookup_normalize(user_id, item_id, utab2, itab2)
    return (uoT.T, ioT.T)
